# baseline (speedup 1.0000x reference)
import functools

import jax
import jax.numpy as jnp
from jax.experimental import pallas as pl


def _chamfer_body(b2_ref, at_ref, d1_ref, d2_ref, *, num_mb, tm):
    mb = pl.program_id(1)
    b2 = b2_ref[0]
    at = at_ref[0]
    x2sq = b2[:, 3:4]
    x1sq = at[3:4, :]
    xynt = jax.lax.dot_general(
        b2[:, 0:3], at[0:3, :], (((1,), (0,)), ((), ())),
        preferred_element_type=jnp.float32,
    )

    part1 = jnp.min(xynt + x2sq, axis=0)

    @pl.when(mb == 0)
    def _():
        d1_ref[0, 0] = part1

    @pl.when((mb > 0) & (mb < num_mb - 1))
    def _():
        d1_ref[0, 0] = jnp.minimum(d1_ref[0, 0], part1)

    @pl.when(mb == num_mb - 1)
    def _():
        d1_ref[0, 0] = jnp.minimum(d1_ref[0, 0], part1) + x1sq[0]

    e = xynt + x1sq
    g = e[:, 0:128]
    for k in range(1, e.shape[1] // 128):
        g = jnp.minimum(g, e[:, k * 128:(k + 1) * 128])
    d2_ref[0] = jnp.min(g, axis=1, keepdims=True) + x2sq


def _chamfer_dists(xyz1, xyz2, *, tm=512, interpret=False):
    B, N, _ = xyz1.shape
    M = xyz2.shape[1]
    num_mb = M // tm
    x1sq = jnp.sum(xyz1 * xyz1, axis=2, keepdims=True)
    x2sq = jnp.sum(xyz2 * xyz2, axis=2, keepdims=True)
    b2 = jnp.concatenate([-2.0 * xyz2, x2sq], axis=2)
    at = jnp.concatenate(
        [jnp.transpose(xyz1, (0, 2, 1)), jnp.transpose(x1sq, (0, 2, 1))],
        axis=1)

    d1, d2 = pl.pallas_call(
        functools.partial(_chamfer_body, num_mb=num_mb, tm=tm),
        grid=(B, num_mb),
        in_specs=[
            pl.BlockSpec((1, tm, 4), lambda b, mb: (b, mb, 0)),
            pl.BlockSpec((1, 4, N), lambda b, mb: (b, 0, 0)),
        ],
        out_specs=[
            pl.BlockSpec((1, 1, N), lambda b, mb: (b, 0, 0)),
            pl.BlockSpec((1, tm, 1), lambda b, mb: (b, mb, 0)),
        ],
        out_shape=[
            jax.ShapeDtypeStruct((B, 1, N), jnp.float32),
            jax.ShapeDtypeStruct((B, M, 1), jnp.float32),
        ],
        interpret=interpret,
    )(b2, at)
    return d1[:, 0, :], d2[:, :, 0]


@jax.jit
def kernel(xyz1, xyz2, weights1, weights2):
    dist1, dist2 = _chamfer_dists(xyz1, xyz2)
    dist1_avg = jnp.sum(dist1 * weights1) / jnp.sum(weights1)
    dist2_avg = jnp.sum(dist2 * weights2) / jnp.sum(weights2)
    return (dist1_avg + dist2_avg) / 2.0


# speedup vs baseline: 1.2136x; 1.2136x over previous
import functools

import jax
import jax.numpy as jnp
from jax.experimental import pallas as pl
from jax.experimental.pallas import tpu as pltpu

_SUB = 512


def _chamfer_body(x1_ref, x2t_ref, d1_ref, d2_ref, acc_ref, *, num_steps, tm):
    step = pl.program_id(1)
    a = x1_ref[0]
    bt = x2t_ref[0]
    x1sq = a[:, 3:4]

    gs = []
    for h in range(tm // _SUB):
        hs = slice(h * _SUB, (h + 1) * _SUB)
        bth = bt[:, hs]
        xyn = jax.lax.dot_general(
            a[:, 0:3], bth[0:3, :], (((1,), (0,)), ((), ())),
            preferred_element_type=jnp.float32,
        )

        d2_ref[0, 0, hs] = jnp.min(xyn + x1sq, axis=0) + bth[3, :]

        x2sq = bth[3:4, :]
        g = xyn[:, 0:128] + x2sq[:, 0:128]
        for k in range(1, _SUB // 128):
            sl = slice(k * 128, (k + 1) * 128)
            g = jnp.minimum(g, xyn[:, sl] + x2sq[:, sl])
        gs.append(g)

    g = gs[0]
    for gh in gs[1:]:
        g = jnp.minimum(g, gh)

    @pl.when(step == 0)
    def _():
        acc_ref[...] = g

    @pl.when(step > 0)
    def _():
        acc_ref[...] = jnp.minimum(acc_ref[...], g)

    @pl.when(step == num_steps - 1)
    def _():
        d1_ref[0, 0] = jnp.min(acc_ref[...], axis=1) + x1sq[:, 0]


def _chamfer_dists(xyz1, xyz2, *, tm=1024, interpret=False):
    B, N, _ = xyz1.shape
    M = xyz2.shape[1]
    num_steps = M // tm
    x1sq = jnp.sum(xyz1 * xyz1, axis=2, keepdims=True)
    a = jnp.concatenate([-2.0 * xyz1, x1sq], axis=2)
    x2t = jnp.transpose(xyz2, (0, 2, 1))
    x2sq = jnp.sum(x2t * x2t, axis=1, keepdims=True)
    bt = jnp.concatenate([x2t, x2sq], axis=1)

    d1, d2 = pl.pallas_call(
        functools.partial(_chamfer_body, num_steps=num_steps, tm=tm),
        grid=(B, num_steps),
        in_specs=[
            pl.BlockSpec((1, N, 4), lambda b, mb: (b, 0, 0)),
            pl.BlockSpec((1, 4, tm), lambda b, mb: (b, 0, mb)),
        ],
        out_specs=[
            pl.BlockSpec((1, 1, N), lambda b, mb: (b, 0, 0)),
            pl.BlockSpec((1, 1, tm), lambda b, mb: (b, 0, mb)),
        ],
        out_shape=[
            jax.ShapeDtypeStruct((B, 1, N), jnp.float32),
            jax.ShapeDtypeStruct((B, 1, M), jnp.float32),
        ],
        scratch_shapes=[pltpu.VMEM((N, 128), jnp.float32)],
        interpret=interpret,
    )(a, bt)
    return d1[:, 0, :], d2[:, 0, :]


@jax.jit
def kernel(xyz1, xyz2, weights1, weights2):
    dist1, dist2 = _chamfer_dists(xyz1, xyz2)
    dist1_avg = jnp.sum(dist1 * weights1) / jnp.sum(weights1)
    dist2_avg = jnp.sum(dist2 * weights2) / jnp.sum(weights2)
    return (dist1_avg + dist2_avg) / 2.0


# speedup vs baseline: 1.3205x; 1.0881x over previous
import functools

import jax
import jax.numpy as jnp
from jax.experimental import pallas as pl
from jax.experimental.pallas import tpu as pltpu

_SUB = 512


def _chamfer_body(x1_ref, x2t_ref, d1_ref, d2_ref, acc_ref, *, num_steps, tm):
    step = pl.program_id(1)
    a = x1_ref[0]
    bt = x2t_ref[0]
    x1sq = a[:, 3:4]

    gs = []
    for h in range(tm // _SUB):
        hs = slice(h * _SUB, (h + 1) * _SUB)
        bth = bt[:, hs]
        xyn = jax.lax.dot_general(
            a[:, 0:3], bth[0:3, :], (((1,), (0,)), ((), ())),
            preferred_element_type=jnp.float32,
        )

        d2_ref[0, 0, hs] = jnp.min(xyn + x1sq, axis=0) + bth[3, :]

        x2sq = bth[3:4, :]
        g = xyn[:, 0:128] + x2sq[:, 0:128]
        for k in range(1, _SUB // 128):
            sl = slice(k * 128, (k + 1) * 128)
            g = jnp.minimum(g, xyn[:, sl] + x2sq[:, sl])
        gs.append(g)

    g = gs[0]
    for gh in gs[1:]:
        g = jnp.minimum(g, gh)

    @pl.when(step == 0)
    def _():
        acc_ref[...] = g

    @pl.when(step > 0)
    def _():
        acc_ref[...] = jnp.minimum(acc_ref[...], g)

    @pl.when(step == num_steps - 1)
    def _():
        d1_ref[0, 0] = jnp.min(acc_ref[...], axis=1) + x1sq[:, 0]


def _chamfer_dists(xyz1, xyz2, *, tm=2048, interpret=False):
    B, N, _ = xyz1.shape
    M = xyz2.shape[1]
    num_steps = M // tm
    x1sq = jnp.sum(xyz1 * xyz1, axis=2, keepdims=True)
    a = jnp.concatenate([-2.0 * xyz1, x1sq], axis=2)
    x2t = jnp.transpose(xyz2, (0, 2, 1))
    x2sq = jnp.sum(x2t * x2t, axis=1, keepdims=True)
    bt = jnp.concatenate([x2t, x2sq], axis=1)

    d1, d2 = pl.pallas_call(
        functools.partial(_chamfer_body, num_steps=num_steps, tm=tm),
        grid=(B, num_steps),
        in_specs=[
            pl.BlockSpec((1, N, 4), lambda b, mb: (b, 0, 0)),
            pl.BlockSpec((1, 4, tm), lambda b, mb: (b, 0, mb)),
        ],
        out_specs=[
            pl.BlockSpec((1, 1, N), lambda b, mb: (b, 0, 0)),
            pl.BlockSpec((1, 1, tm), lambda b, mb: (b, 0, mb)),
        ],
        out_shape=[
            jax.ShapeDtypeStruct((B, 1, N), jnp.float32),
            jax.ShapeDtypeStruct((B, 1, M), jnp.float32),
        ],
        scratch_shapes=[pltpu.VMEM((N, 128), jnp.float32)],
        interpret=interpret,
    )(a, bt)
    return d1[:, 0, :], d2[:, 0, :]


@jax.jit
def kernel(xyz1, xyz2, weights1, weights2):
    dist1, dist2 = _chamfer_dists(xyz1, xyz2)
    dist1_avg = jnp.sum(dist1 * weights1) / jnp.sum(weights1)
    dist2_avg = jnp.sum(dist2 * weights2) / jnp.sum(weights2)
    return (dist1_avg + dist2_avg) / 2.0
